# baseline (speedup 1.0000x reference)
import functools

import numpy as np
import jax
import jax.numpy as jnp
from jax import lax
from jax.experimental import pallas as pl
from jax.experimental.pallas import tpu as pltpu, tpu_sc as plsc

ROWS = 4096
COLS = 26
VOCAB = 1000
NUM_WORKERS = 32
PLANES_PER_WORKER = ROWS // NUM_WORKERS
L = 16
P = 2
GROUPS = PLANES_PER_WORKER // P
NBATCH = (P * COLS + L - 1) // L


def _batch_consts():
    iota = lax.iota(jnp.int32, L)
    out = []
    for k in range(NBATCH):
        r = iota + k * L
        mask = (r < P * COLS) if (k + 1) * L > P * COLS else None
        rc = jnp.minimum(r, P * COLS - 1)
        out.append((rc // COLS, rc % COLS, mask))
    return out


def _body(x_hbm, zeros_hbm, out_hbm, buf0, buf1, idx_v, sav, sem0, sem1):
    wid = lax.axis_index("c") * 16 + lax.axis_index("s")
    base = wid * PLANES_PER_WORKER

    bufs = (buf0, buf1)
    sems = (sem0, sem1)

    pltpu.async_copy(zeros_hbm, buf0, sem0)
    pltpu.async_copy(zeros_hbm, buf1, sem1)

    pltpu.sync_copy(x_hbm.at[pl.ds(base, PLANES_PER_WORKER)], idx_v)

    consts = _batch_consts()
    ones = jnp.full((L,), 1.0, jnp.float32)
    zeros_v = jnp.zeros((L,), jnp.float32)
    zeros_i = jnp.zeros((L,), jnp.int32)

    for i in range(2 * NBATCH):
        sav[i, :] = zeros_i

    def step(g, carry):
        for b in range(2):
            buf, sem = bufs[b], sems[b]
            first_plane = (2 * g + b) * P
            pltpu.make_async_copy(zeros_hbm, buf, sem).wait()
            for k, (poff, rows, mask) in enumerate(consts):
                slot = b * NBATCH + k
                plsc.store_scatter(buf, [poff, rows, sav[slot, :]],
                                   zeros_v, mask=mask)
                cols = plsc.load_gather(idx_v, [first_plane + poff, rows],
                                        mask=mask)
                if mask is not None:
                    cols = jnp.where(mask, cols, 0)
                plsc.store_scatter(buf, [poff, rows, cols], ones, mask=mask)
                sav[slot, :] = cols
            pltpu.async_copy(buf, out_hbm.at[pl.ds(base + first_plane, P)],
                             sem)
        return carry

    lax.fori_loop(0, GROUPS // 2, step, 0)

    pltpu.make_async_copy(zeros_hbm, buf0, sem0).wait()
    pltpu.make_async_copy(zeros_hbm, buf1, sem1).wait()


_onehot_sc = functools.partial(
    pl.kernel,
    out_type=jax.ShapeDtypeStruct((ROWS, COLS, VOCAB), jnp.float32),
    mesh=plsc.VectorSubcoreMesh(core_axis_name="c", subcore_axis_name="s"),
    compiler_params=pltpu.CompilerParams(
        use_tc_tiling_on_sc=False, needs_layout_passes=False),
    scratch_types=[
        pltpu.VMEM((P, COLS, VOCAB), jnp.float32),
        pltpu.VMEM((P, COLS, VOCAB), jnp.float32),
        pltpu.VMEM((PLANES_PER_WORKER, COLS), jnp.int32),
        pltpu.VMEM((2 * NBATCH, L), jnp.int32),
        pltpu.SemaphoreType.DMA,
        pltpu.SemaphoreType.DMA,
    ],
)(_body)


def kernel(x):
    zeros = jnp.zeros((P, COLS, VOCAB), jnp.float32)
    return _onehot_sc(x, zeros)
